# baseline (speedup 1.0000x reference)
import jax
import jax.numpy as jnp
from jax import lax
from jax.experimental import pallas as pl
from jax.experimental.pallas import tpu as pltpu
from jax.experimental.pallas import tpu_sc as plsc

_DIM = 16
_NC = 2
_NS = 16
_NW = _NC * _NS
_NBUF = 8
_D = 4


def _gather_body(idx_hbm, table_hbm, out_hbm, idx_v, bufg, bufs, *sems):
    gsem = sems[:_NBUF]
    ssem = sems[_NBUF:]
    nj = idx_v.shape[0]
    ki = idx_v.shape[1]
    hist = ki // 2
    wid = lax.axis_index("s") * _NC + lax.axis_index("c")
    wr0 = wid * 2 * nj

    pltpu.sync_copy(idx_hbm.at[pl.ds(wid * nj, nj), :], idx_v)

    def fire_gather(b, j):
        pltpu.async_copy(table_hbm.at[idx_v.at[j]], bufg.at[b], gsem[b])

    def wait_gather(b, j):
        pltpu.make_async_copy(table_hbm.at[idx_v.at[j]], bufg.at[b], gsem[b]).wait()

    def fire_scatter(b, j):
        pltpu.async_copy(
            bufs.at[b], out_hbm.at[pl.ds(wr0 + 2 * j, 2), :, :], ssem[b])

    def wait_scatter(b, j):
        pltpu.make_async_copy(
            bufs.at[b], out_hbm.at[pl.ds(wr0 + 2 * j, 2), :, :], ssem[b]).wait()

    def relay(b):
        for i in range(ki):
            bufs[b, i // hist, i % hist, :] = plsc.bitcast(
                bufg[b, i, :], jnp.bfloat16)

    for j in range(_D):
        fire_gather(j % _NBUF, j)

    def chunk(j, b, first=False, last=False):
        wait_gather(b, j)
        if not first:
            wait_scatter(b, j - _NBUF)
        relay(b)
        fire_scatter(b, j)
        if not last:
            fire_gather((b + _D) % _NBUF, j + _D)

    for b in range(_NBUF):
        chunk(b, b, first=True)

    @pl.loop(_NBUF, nj - _NBUF, step=_NBUF)
    def _r(j0):
        for b in range(_NBUF):
            chunk(j0 + b, b)

    for b in range(_NBUF):
        j = nj - _NBUF + b
        chunk(j, b, last=(j + _D >= nj))
    for b in range(_NBUF):
        wait_scatter(b, nj - _NBUF + b)


@jax.jit
def _run(idx, table32):
    nrow2, ki = idx.shape
    hist = ki // 2
    nj = nrow2 // _NW
    f = pl.kernel(
        _gather_body,
        out_type=jax.ShapeDtypeStruct((2 * nrow2, hist, 4 * _DIM // 2),
                                      jnp.bfloat16),
        mesh=plsc.VectorSubcoreMesh(core_axis_name="c", subcore_axis_name="s"),
        scratch_types=[
            pltpu.VMEM((nj, ki), jnp.int32),
            pltpu.VMEM((_NBUF, ki, _DIM), jnp.int32),
            pltpu.VMEM((_NBUF, 2, hist, 2 * _DIM), jnp.bfloat16),
        ] + [pltpu.SemaphoreType.DMA] * (2 * _NBUF),
        compiler_params=pltpu.CompilerParams(
            use_tc_tiling_on_sc=False, needs_layout_passes=False),
    )
    return f(idx, table32)


def kernel(input, weight):
    b, h = input.shape
    assert b % _NW == 0 and (b // _NW) % _NBUF == 0
    nrows, dim = weight.shape
    w32 = jax.lax.bitcast_convert_type(
        weight.reshape(nrows, dim // 2, 2), jnp.int32)
    idx2 = input.astype(jnp.int32).reshape(b // 2, 2 * h)
    return _run(idx2, w32)
